# baseline (speedup 1.0000x reference)
import functools

import jax
import jax.numpy as jnp
from jax import lax
from jax.experimental import pallas as pl
from jax.experimental.pallas import tpu as pltpu
from jax.experimental.pallas import tpu_sc as plsc

NC, NS = 2, 16
NW = NC * NS
DM = 128


def _rup(n, m):
    return (n + m - 1) // m * m


def _mlp(p, xs, *, residual=False, e_new=False, proj=None, br=1024):
    n = xs[0].shape[0]
    dins = [x.shape[1] for x in xs]
    k = len(xs)
    dout = proj[0].shape[1] if proj is not None else DM

    def body(*refs):
        xrefs = refs[:k]
        w1, b1, w2, b2, g, bn = refs[k:k + 6]
        pos = k + 6
        if proj is not None:
            wo, bo = refs[pos:pos + 2]
            pos += 2
        outs = refs[pos:]
        bf = jnp.bfloat16
        acc = None
        off = 0
        for i in range(k):
            part = jnp.dot(xrefs[i][...].astype(bf),
                           w1[off:off + dins[i], :].astype(bf),
                           preferred_element_type=jnp.float32)
            acc = part if acc is None else acc + part
            off += dins[i]
        h = acc + b1[...]
        h = h * jax.nn.sigmoid(h)
        y = jnp.dot(h.astype(bf), w2[...].astype(bf),
                    preferred_element_type=jnp.float32) + b2[...]
        mu = jnp.mean(y, -1, keepdims=True)
        yc = y - mu
        var = jnp.mean(yc * yc, -1, keepdims=True)
        m = yc * lax.rsqrt(var + 1e-5) * g[...] + bn[...]
        r = xrefs[0][...] + m if residual else m
        if proj is not None:
            r = jnp.dot(r, wo[...], preferred_element_type=jnp.float32) + bo[...]
        outs[0][...] = r
        if e_new:
            outs[1][...] = xrefs[-1][...] + m

    in_specs = [pl.BlockSpec((br, d), lambda i: (i, 0)) for d in dins]
    w_args = [p["W1"], p["b1"].reshape(1, DM), p["W2"], p["b2"].reshape(1, DM),
              p["g"].reshape(1, DM), p["bn"].reshape(1, DM)]
    for w in w_args:
        in_specs.append(pl.BlockSpec(w.shape, lambda i: (0, 0)))
    args = list(xs) + w_args
    if proj is not None:
        wo, bo = proj
        args += [wo, bo.reshape(1, -1)]
        in_specs.append(pl.BlockSpec(wo.shape, lambda i: (0, 0)))
        in_specs.append(pl.BlockSpec((1, dout), lambda i: (0, 0)))
    out_shape = [jax.ShapeDtypeStruct((n, dout), jnp.float32)]
    out_specs = [pl.BlockSpec((br, dout), lambda i: (i, 0))]
    if e_new:
        out_shape.append(jax.ShapeDtypeStruct((n, DM), jnp.float32))
        out_specs.append(pl.BlockSpec((br, DM), lambda i: (i, 0)))
    res = pl.pallas_call(
        body,
        grid=(pl.cdiv(n, br),),
        in_specs=in_specs,
        out_specs=out_specs,
        out_shape=out_shape,
    )(*args)
    return res if e_new else res[0]


def _sc_gather2(ta, tb, ia3, ib3):
    cpw = ia3.shape[1]
    nd = 4
    nbody, ntail = divmod(cpw, nd)
    e_pad = NW * cpw * 128
    mesh = plsc.VectorSubcoreMesh(core_axis_name="c", subcore_axis_name="s",
                                  num_cores=NC, num_subcores=NS)

    @functools.partial(
        pl.kernel,
        out_type=(jax.ShapeDtypeStruct((e_pad, DM), jnp.float32),
                  jax.ShapeDtypeStruct((e_pad, DM), jnp.float32)),
        mesh=mesh,
        scratch_types=[pltpu.VMEM((2, cpw, 128), jnp.int32),
                       pltpu.VMEM((nd, 128, DM), jnp.float32)]
                      + [pltpu.SemaphoreType.DMA] * (2 * nd),
    )
    def k(ta_h, tb_h, ia_h, ib_h, oa_h, ob_h, idxv, rows, *sems):
        gsems, osems = sems[:nd], sems[nd:]
        wid = lax.axis_index("s") * NC + lax.axis_index("c")
        pltpu.sync_copy(ia_h.at[wid], idxv.at[0])
        pltpu.sync_copy(ib_h.at[wid], idxv.at[1])
        for t in range(2):
            tbl = ta_h if t == 0 else tb_h
            out = oa_h if t == 0 else ob_h

            def win(j0, nwin, tbl=tbl, out=out, t=t):
                gds = [pltpu.async_copy(tbl.at[idxv.at[t, j0 + s]],
                                        rows.at[s], gsems[s])
                       for s in range(nwin)]
                ods = []
                for s in range(nwin):
                    gds[s].wait()
                    ods.append(pltpu.async_copy(
                        rows.at[s],
                        out.at[pl.ds((wid * cpw + j0 + s) * 128, 128)],
                        osems[s]))
                for s in range(nwin):
                    ods[s].wait()

            def body(i, carry):
                win(i * nd, nd)
                return carry

            lax.fori_loop(0, nbody, body, 0)
            if ntail:
                win(nbody * nd, ntail)

    return k(ta, tb, ia3, ib3)


def _sc_scatter(m2, idx2, n_nodes, n_passes):
    cpt = idx2.shape[1]
    r_al = _rup(-(-n_nodes // (NC * n_passes)), 128)
    r_pad = _rup(r_al + 1, NS * 8)
    zr = r_pad // NS
    zc, zrem = divmod(zr, 128)
    wb = r_al // NS
    nd = 4
    nbody, ntail = divmod(cpt, nd)
    mesh = plsc.VectorSubcoreMesh(core_axis_name="c", subcore_axis_name="s",
                                  num_cores=NC, num_subcores=NS)

    @functools.partial(
        pl.kernel,
        out_type=jax.ShapeDtypeStruct((n_nodes, DM), jnp.float32),
        mesh=mesh,
        scratch_types=[pltpu.VMEM((cpt, 128), jnp.int32),
                       pltpu.VMEM((cpt, 128), jnp.int32),
                       pltpu.VMEM((nd, 128, DM), jnp.float32),
                       pltpu.VMEM_SHARED((r_pad, DM), jnp.float32)]
                      + [pltpu.SemaphoreType.DMA] * (2 * nd + 1),
    )
    def k(m_h, i_h, out_h, idxb, lidxb, rows, shared, *sems):
        lsems, ssems, zsem = sems[:nd], sems[nd:2 * nd], sems[2 * nd]
        cid = lax.axis_index("c")
        sid = lax.axis_index("s")
        pltpu.sync_copy(i_h.at[sid], idxb)
        zb = sid * zr
        for pss in range(n_passes):
            rs = jnp.minimum((cid * n_passes + pss) * r_al, n_nodes - r_al)

            def zrow(rr, carry):
                for j in range(8):
                    rows[0, rr, pl.ds(j * 16, 16)] = jnp.zeros((16,), jnp.float32)
                return carry
            lax.fori_loop(0, 128, zrow, 0)
            zds = [pltpu.async_copy(rows.at[0],
                                    shared.at[pl.ds(zb + z * 128, 128)], zsem)
                   for z in range(zc)]
            if zrem:
                zds.append(pltpu.async_copy(
                    rows.at[0, pl.ds(0, zrem)],
                    shared.at[pl.ds(zb + zc * 128, zrem)], zsem))

            def lix(c, carry):
                for j in range(8):
                    v = idxb[c, pl.ds(j * 16, 16)]
                    li = v - rs
                    okm = (li >= 0) & (li < r_al)
                    lidxb[c, pl.ds(j * 16, 16)] = jnp.where(okm, li, r_al)
                return carry
            lax.fori_loop(0, cpt, lix, 0)
            for d in zds:
                d.wait()
            plsc.subcore_barrier()

            def win(j0, nwin):
                lds = [pltpu.async_copy(
                           m_h.at[pl.ds((sid * cpt + j0 + s) * 128, 128)],
                           rows.at[s], lsems[s])
                       for s in range(nwin)]
                sds = []
                for s in range(nwin):
                    lds[s].wait()
                    sds.append(pltpu.async_copy(
                        rows.at[s], shared.at[lidxb.at[j0 + s]],
                        ssems[s], add=True))
                for s in range(nwin):
                    sds[s].wait()

            def body(i, carry):
                win(i * nd, nd)
                return carry

            lax.fori_loop(0, nbody, body, 0)
            if ntail:
                win(nbody * nd, ntail)
            plsc.subcore_barrier()
            pltpu.sync_copy(shared.at[pl.ds(sid * wb, wb)],
                            out_h.at[pl.ds(rs + sid * wb, wb)])
            plsc.subcore_barrier()

    return k(m2, idx2)


def _pad_rows(a, n_pad):
    e = a.shape[0]
    if e == n_pad:
        return a
    return jnp.concatenate(
        [a, jnp.zeros((n_pad - e,) + a.shape[1:], a.dtype)], axis=0)


def _pad_idx(idx, n_pad, fill):
    e = idx.shape[0]
    if e != n_pad:
        idx = jnp.concatenate(
            [idx, jnp.full((n_pad - e,), fill, jnp.int32)], axis=0)
    return idx


def _pad_idx_g(idx, n_pad):
    return _pad_idx(idx, n_pad, 0).reshape(NW, -1, 128)


def _pad_idx_s(idx, n_pad):
    return _pad_idx(idx, n_pad, 1 << 30).reshape(NS, -1, 128)


def kernel(x, mgroupdef, e2h_edge_index, h2h_edge_index, h2e_edge_index,
           e2h_edge_attr, h2h_edge_attr, h2e_edge_attr,
           era_latlons, h_latlons, params):
    p = params
    bs = x.shape[0]
    n_era = x.shape[2]
    n_h = h_latlons.shape[0]
    e_e2h = e2h_edge_index.shape[1]
    e_h2h = h2h_edge_index.shape[1]
    e_h2e = h2e_edge_index.shape[1]
    e2h_pad = _rup(e_e2h, NW * 128)
    h2h_pad = _rup(e_h2h, NW * 128)
    h2e_pad = _rup(e_h2e, NW * 128)

    x_flat = jnp.transpose(x, (0, 2, 1, 3)).reshape(bs * n_era, -1)

    src = _mlp(p["fm_src"], [x_flat, era_latlons, p["era_trainable"]])
    dst = _mlp(p["fm_dst"], [h_latlons, p["h_trainable"]])
    e_fm = _mlp(p["fm_edge"], [_pad_rows(e2h_edge_attr, e2h_pad),
                               _pad_rows(p["e2h_trainable"], e2h_pad)])

    e2h_s = _pad_idx_g(e2h_edge_index[0], e2h_pad)
    e2h_d = _pad_idx_g(e2h_edge_index[1], e2h_pad)
    e2h_dscat = _pad_idx_s(e2h_edge_index[1], e2h_pad)
    gs, gd = _sc_gather2(src, dst, e2h_s, e2h_d)
    m = _mlp(p["fm_msg"], [gs, gd, e_fm])
    agg = _sc_scatter(m, e2h_dscat, n_h, 1)
    x_latent = _mlp(p["fm_node"], [dst, agg], residual=True)

    e_pr = _mlp(p["proc_edge"], [_pad_rows(h2h_edge_attr, h2h_pad),
                                 _pad_rows(p["h2h_trainable"], h2h_pad)])
    h2h_s = _pad_idx_g(h2h_edge_index[0], h2h_pad)
    h2h_d = _pad_idx_g(h2h_edge_index[1], h2h_pad)
    h2h_dscat = _pad_idx_s(h2h_edge_index[1], h2h_pad)

    gs, gd = _sc_gather2(x_latent, x_latent, h2h_s, h2h_d)
    m0, e_pr = _mlp(p["proc_msg_0"], [gs, gd, e_pr], e_new=True)
    agg = _sc_scatter(m0, h2h_dscat, n_h, 1)
    x_latent = _mlp(p["proc_node_0"], [x_latent, agg], residual=True)

    gs, gd = _sc_gather2(x_latent, x_latent, h2h_s, h2h_d)
    m1 = _mlp(p["proc_msg_1"], [gs, gd, e_pr])
    agg = _sc_scatter(m1, h2h_dscat, n_h, 1)
    x_latent = _mlp(p["proc_node_1"], [x_latent, agg], residual=True)

    e_bm = _mlp(p["bm_edge"], [_pad_rows(h2e_edge_attr, h2e_pad),
                               _pad_rows(p["h2e_trainable"], h2e_pad)])
    h2e_s = _pad_idx_g(h2e_edge_index[0], h2e_pad)
    h2e_d = _pad_idx_g(h2e_edge_index[1], h2e_pad)
    h2e_dscat = _pad_idx_s(h2e_edge_index[1], h2e_pad)
    gs, gd = _sc_gather2(x_latent, src, h2e_s, h2e_d)
    m2 = _mlp(p["bm_msg"], [gs, gd, e_bm])
    agg = _sc_scatter(m2, h2e_dscat, n_era, 5)
    out = _mlp(p["bm_node"], [src, agg], residual=True,
               proj=(p["bm_out_W"], p["bm_out_b"]))
    return out.reshape(bs, n_era, -1)


# speedup vs baseline: 1.1393x; 1.1393x over previous
import functools

import jax
import jax.numpy as jnp
from jax import lax
from jax.experimental import pallas as pl
from jax.experimental.pallas import tpu as pltpu
from jax.experimental.pallas import tpu_sc as plsc

NC, NS = 2, 16
NW = NC * NS
DM = 128


def _rup(n, m):
    return (n + m - 1) // m * m


def _mlp(p, xs, *, residual=False, e_new=False, proj=None, br=1024, offs=None):
    n = xs[0].shape[0]
    dins = [x.shape[1] for x in xs]
    k = len(xs)
    dout = proj[0].shape[1] if proj is not None else DM

    def body(*refs):
        xrefs = refs[:k]
        w1, b1, w2, b2, g, bn = refs[k:k + 6]
        pos = k + 6
        if proj is not None:
            wo, bo = refs[pos:pos + 2]
            pos += 2
        outs = refs[pos:]
        bf = jnp.bfloat16
        acc = None
        off = 0
        for i in range(k):
            o = offs[i] if offs is not None else off
            part = jnp.dot(xrefs[i][...].astype(bf),
                           w1[o:o + dins[i], :].astype(bf),
                           preferred_element_type=jnp.float32)
            acc = part if acc is None else acc + part
            off += dins[i]
        h = acc + b1[...]
        h = h * jax.nn.sigmoid(h)
        y = jnp.dot(h.astype(bf), w2[...].astype(bf),
                    preferred_element_type=jnp.float32) + b2[...]
        mu = jnp.mean(y, -1, keepdims=True)
        yc = y - mu
        var = jnp.mean(yc * yc, -1, keepdims=True)
        m = yc * lax.rsqrt(var + 1e-5) * g[...] + bn[...]
        r = xrefs[0][...] + m if residual else m
        if proj is not None:
            r = jnp.dot(r, wo[...], preferred_element_type=jnp.float32) + bo[...]
        outs[0][...] = r
        if e_new:
            outs[1][...] = xrefs[-1][...] + m

    in_specs = [pl.BlockSpec((br, d), lambda i: (i, 0)) for d in dins]
    w_args = [p["W1"], p["b1"].reshape(1, DM), p["W2"], p["b2"].reshape(1, DM),
              p["g"].reshape(1, DM), p["bn"].reshape(1, DM)]
    for w in w_args:
        in_specs.append(pl.BlockSpec(w.shape, lambda i: (0, 0)))
    args = list(xs) + w_args
    if proj is not None:
        wo, bo = proj
        args += [wo, bo.reshape(1, -1)]
        in_specs.append(pl.BlockSpec(wo.shape, lambda i: (0, 0)))
        in_specs.append(pl.BlockSpec((1, dout), lambda i: (0, 0)))
    out_shape = [jax.ShapeDtypeStruct((n, dout), jnp.float32)]
    out_specs = [pl.BlockSpec((br, dout), lambda i: (i, 0))]
    if e_new:
        out_shape.append(jax.ShapeDtypeStruct((n, DM), jnp.float32))
        out_specs.append(pl.BlockSpec((br, DM), lambda i: (i, 0)))
    res = pl.pallas_call(
        body,
        grid=(pl.cdiv(n, br),),
        in_specs=in_specs,
        out_specs=out_specs,
        out_shape=out_shape,
    )(*args)
    return res if e_new else res[0]


def _sc_gather2(ta, tb, ia3, ib3):
    cpw = ia3.shape[1]
    nd = 4
    nbody, ntail = divmod(cpw, nd)
    e_pad = NW * cpw * 128
    mesh = plsc.VectorSubcoreMesh(core_axis_name="c", subcore_axis_name="s",
                                  num_cores=NC, num_subcores=NS)

    @functools.partial(
        pl.kernel,
        out_type=(jax.ShapeDtypeStruct((e_pad, DM), jnp.float32),
                  jax.ShapeDtypeStruct((e_pad, DM), jnp.float32)),
        mesh=mesh,
        scratch_types=[pltpu.VMEM((2, cpw, 128), jnp.int32),
                       pltpu.VMEM((nd, 128, DM), jnp.float32)]
                      + [pltpu.SemaphoreType.DMA] * (2 * nd),
    )
    def k(ta_h, tb_h, ia_h, ib_h, oa_h, ob_h, idxv, rows, *sems):
        gsems, osems = sems[:nd], sems[nd:]
        wid = lax.axis_index("s") * NC + lax.axis_index("c")
        pltpu.sync_copy(ia_h.at[wid], idxv.at[0])
        pltpu.sync_copy(ib_h.at[wid], idxv.at[1])
        for t in range(2):
            tbl = ta_h if t == 0 else tb_h
            out = oa_h if t == 0 else ob_h

            def win(j0, nwin, tbl=tbl, out=out, t=t):
                gds = [pltpu.async_copy(tbl.at[idxv.at[t, j0 + s]],
                                        rows.at[s], gsems[s])
                       for s in range(nwin)]
                ods = []
                for s in range(nwin):
                    gds[s].wait()
                    ods.append(pltpu.async_copy(
                        rows.at[s],
                        out.at[pl.ds((wid * cpw + j0 + s) * 128, 128)],
                        osems[s]))
                for s in range(nwin):
                    ods[s].wait()

            def body(i, carry):
                win(i * nd, nd)
                return carry

            lax.fori_loop(0, nbody, body, 0)
            if ntail:
                win(nbody * nd, ntail)

    return k(ta, tb, ia3, ib3)


def _sc_scatter(m2, idx2, n_nodes, n_passes):
    cpt = idx2.shape[1]
    r_al = _rup(-(-n_nodes // (NC * n_passes)), 128)
    r_pad = _rup(r_al + 1, NS * 8)
    zr = r_pad // NS
    zc, zrem = divmod(zr, 128)
    wb = r_al // NS
    nd = 4
    nbody, ntail = divmod(cpt, nd)
    mesh = plsc.VectorSubcoreMesh(core_axis_name="c", subcore_axis_name="s",
                                  num_cores=NC, num_subcores=NS)

    @functools.partial(
        pl.kernel,
        out_type=jax.ShapeDtypeStruct((n_nodes, DM), jnp.float32),
        mesh=mesh,
        scratch_types=[pltpu.VMEM((cpt, 128), jnp.int32),
                       pltpu.VMEM((cpt, 128), jnp.int32),
                       pltpu.VMEM((nd, 128, DM), jnp.float32),
                       pltpu.VMEM_SHARED((r_pad, DM), jnp.float32)]
                      + [pltpu.SemaphoreType.DMA] * (2 * nd + 1),
    )
    def k(m_h, i_h, out_h, idxb, lidxb, rows, shared, *sems):
        lsems, ssems, zsem = sems[:nd], sems[nd:2 * nd], sems[2 * nd]
        cid = lax.axis_index("c")
        sid = lax.axis_index("s")
        pltpu.sync_copy(i_h.at[sid], idxb)
        zb = sid * zr
        for pss in range(n_passes):
            rs = jnp.minimum((cid * n_passes + pss) * r_al, n_nodes - r_al)

            def zrow(rr, carry):
                for j in range(8):
                    rows[0, rr, pl.ds(j * 16, 16)] = jnp.zeros((16,), jnp.float32)
                return carry
            lax.fori_loop(0, 128, zrow, 0)
            zds = [pltpu.async_copy(rows.at[0],
                                    shared.at[pl.ds(zb + z * 128, 128)], zsem)
                   for z in range(zc)]
            if zrem:
                zds.append(pltpu.async_copy(
                    rows.at[0, pl.ds(0, zrem)],
                    shared.at[pl.ds(zb + zc * 128, zrem)], zsem))

            def lix(c, carry):
                for j in range(8):
                    v = idxb[c, pl.ds(j * 16, 16)]
                    li = v - rs
                    okm = (li >= 0) & (li < r_al)
                    lidxb[c, pl.ds(j * 16, 16)] = jnp.where(okm, li, r_al)
                return carry
            lax.fori_loop(0, cpt, lix, 0)
            for d in zds:
                d.wait()
            plsc.subcore_barrier()

            def win(j0, nwin):
                lds = [pltpu.async_copy(
                           m_h.at[pl.ds((sid * cpt + j0 + s) * 128, 128)],
                           rows.at[s], lsems[s])
                       for s in range(nwin)]
                sds = []
                for s in range(nwin):
                    lds[s].wait()
                    sds.append(pltpu.async_copy(
                        rows.at[s], shared.at[lidxb.at[j0 + s]],
                        ssems[s], add=True))
                for s in range(nwin):
                    sds[s].wait()

            def body(i, carry):
                win(i * nd, nd)
                return carry

            lax.fori_loop(0, nbody, body, 0)
            if ntail:
                win(nbody * nd, ntail)
            plsc.subcore_barrier()
            pltpu.sync_copy(shared.at[pl.ds(sid * wb, wb)],
                            out_h.at[pl.ds(rs + sid * wb, wb)])
            plsc.subcore_barrier()

    return k(m2, idx2)


def _pad_rows(a, n_pad):
    e = a.shape[0]
    if e == n_pad:
        return a
    return jnp.concatenate(
        [a, jnp.zeros((n_pad - e,) + a.shape[1:], a.dtype)], axis=0)


def _pad_idx(idx, n_pad, fill):
    e = idx.shape[0]
    if e != n_pad:
        idx = jnp.concatenate(
            [idx, jnp.full((n_pad - e,), fill, jnp.int32)], axis=0)
    return idx


def _pad_idx_g(idx, n_pad):
    return _pad_idx(idx, n_pad, 0).reshape(NW, -1, 128)


def _pad_idx_s(idx, n_pad):
    return _pad_idx(idx, n_pad, 1 << 30).reshape(NS, -1, 128)


def _halves(n_e):
    e1 = min(_rup(n_e - n_e // 2, NW * 128), n_e)
    return [(0, e1, _rup(e1, NW * 128)), (e1, n_e - e1, _rup(n_e - e1, NW * 128))]


def _edge_idx(edge_index, halves):
    out = []
    for lo, cnt, pad in halves:
        s = lax.dynamic_slice_in_dim(edge_index[0], lo, cnt)
        d = lax.dynamic_slice_in_dim(edge_index[1], lo, cnt)
        out.append((_pad_idx_g(s, pad), _pad_idx_g(d, pad), _pad_idx_s(d, pad)))
    return out


def _mp_half(msg_p, ta, tb, e_half, idx3, n_dst, n_passes, e_new=False):
    gi_s, gi_d, si_d = idx3
    gs, gd = _sc_gather2(ta, tb, gi_s, gi_d)
    if e_new:
        m, e_out = _mlp(msg_p, [gs, gd, e_half], e_new=True)
    else:
        m = _mlp(msg_p, [gs, gd, e_half])
        e_out = None
    agg = _sc_scatter(m, si_d, n_dst, n_passes)
    return agg, e_out


def kernel(x, mgroupdef, e2h_edge_index, h2h_edge_index, h2e_edge_index,
           e2h_edge_attr, h2h_edge_attr, h2e_edge_attr,
           era_latlons, h_latlons, params):
    p = params
    bs = x.shape[0]
    n_era = x.shape[2]
    n_h = h_latlons.shape[0]
    e2h_h = _halves(e2h_edge_index.shape[1])
    h2h_h = _halves(h2h_edge_index.shape[1])
    h2e_h = _halves(h2e_edge_index.shape[1])

    def enc_edges(attr, tr, pe, halves):
        es = []
        for lo, cnt, pad in halves:
            a = _pad_rows(lax.dynamic_slice_in_dim(attr, lo, cnt), pad)
            t = _pad_rows(lax.dynamic_slice_in_dim(tr, lo, cnt), pad)
            es.append(_mlp(pe, [a, t]))
        return es

    x_flat = jnp.transpose(x, (0, 2, 1, 3)).reshape(bs * n_era, -1)

    src = _mlp(p["fm_src"], [x_flat, era_latlons, p["era_trainable"]])
    dst = _mlp(p["fm_dst"], [h_latlons, p["h_trainable"]])
    e_fm = enc_edges(e2h_edge_attr, p["e2h_trainable"], p["fm_edge"], e2h_h)
    e_pr = enc_edges(h2h_edge_attr, p["h2h_trainable"], p["proc_edge"], h2h_h)
    e_bm = enc_edges(h2e_edge_attr, p["h2e_trainable"], p["bm_edge"], h2e_h)

    nodeoffs = [0, DM, DM]

    e2h_i = _edge_idx(e2h_edge_index, e2h_h)
    agg0, _ = _mp_half(p["fm_msg"], src, dst, e_fm[0], e2h_i[0], n_h, 1)
    agg1, _ = _mp_half(p["fm_msg"], src, dst, e_fm[1], e2h_i[1], n_h, 1)
    x_latent = _mlp(p["fm_node"], [dst, agg0, agg1], residual=True,
                    offs=nodeoffs)

    h2h_i = _edge_idx(h2h_edge_index, h2h_h)
    agg0, e0 = _mp_half(p["proc_msg_0"], x_latent, x_latent, e_pr[0],
                        h2h_i[0], n_h, 1, e_new=True)
    agg1, e1 = _mp_half(p["proc_msg_0"], x_latent, x_latent, e_pr[1],
                        h2h_i[1], n_h, 1, e_new=True)
    x_latent = _mlp(p["proc_node_0"], [x_latent, agg0, agg1], residual=True,
                    offs=nodeoffs)
    agg0, _ = _mp_half(p["proc_msg_1"], x_latent, x_latent, e0, h2h_i[0], n_h, 1)
    agg1, _ = _mp_half(p["proc_msg_1"], x_latent, x_latent, e1, h2h_i[1], n_h, 1)
    x_latent = _mlp(p["proc_node_1"], [x_latent, agg0, agg1], residual=True,
                    offs=nodeoffs)

    h2e_i = _edge_idx(h2e_edge_index, h2e_h)
    agg0, _ = _mp_half(p["bm_msg"], x_latent, src, e_bm[0], h2e_i[0], n_era, 5)
    agg1, _ = _mp_half(p["bm_msg"], x_latent, src, e_bm[1], h2e_i[1], n_era, 5)
    out = _mlp(p["bm_node"], [src, agg0, agg1], residual=True,
               proj=(p["bm_out_W"], p["bm_out_b"]), offs=nodeoffs)
    return out.reshape(bs, n_era, -1)


# speedup vs baseline: 1.1409x; 1.0015x over previous
import functools

import jax
import jax.numpy as jnp
from jax import lax
from jax.experimental import pallas as pl
from jax.experimental.pallas import tpu as pltpu
from jax.experimental.pallas import tpu_sc as plsc

NC, NS = 2, 16
NW = NC * NS
DM = 128


def _rup(n, m):
    return (n + m - 1) // m * m


def _mlp(p, xs, *, residual=False, e_new=False, proj=None, br=1024, offs=None):
    n = xs[0].shape[0]
    dins = [x.shape[1] for x in xs]
    k = len(xs)
    dout = proj[0].shape[1] if proj is not None else DM

    def body(*refs):
        xrefs = refs[:k]
        w1, b1, w2, b2, g, bn = refs[k:k + 6]
        pos = k + 6
        if proj is not None:
            wo, bo = refs[pos:pos + 2]
            pos += 2
        outs = refs[pos:]
        bf = jnp.bfloat16
        acc = None
        off = 0
        for i in range(k):
            o = offs[i] if offs is not None else off
            part = jnp.dot(xrefs[i][...].astype(bf),
                           w1[o:o + dins[i], :].astype(bf),
                           preferred_element_type=jnp.float32)
            acc = part if acc is None else acc + part
            off += dins[i]
        h = acc + b1[...]
        h = h * jax.nn.sigmoid(h)
        y = jnp.dot(h.astype(bf), w2[...].astype(bf),
                    preferred_element_type=jnp.float32) + b2[...]
        mu = jnp.mean(y, -1, keepdims=True)
        yc = y - mu
        var = jnp.mean(yc * yc, -1, keepdims=True)
        m = yc * lax.rsqrt(var + 1e-5) * g[...] + bn[...]
        r = xrefs[0][...] + m if residual else m
        if proj is not None:
            r = jnp.dot(r, wo[...], preferred_element_type=jnp.float32) + bo[...]
        outs[0][...] = r
        if e_new:
            outs[1][...] = xrefs[-1][...] + m

    in_specs = [pl.BlockSpec((br, d), lambda i: (i, 0)) for d in dins]
    w_args = [p["W1"], p["b1"].reshape(1, DM), p["W2"], p["b2"].reshape(1, DM),
              p["g"].reshape(1, DM), p["bn"].reshape(1, DM)]
    for w in w_args:
        in_specs.append(pl.BlockSpec(w.shape, lambda i: (0, 0)))
    args = list(xs) + w_args
    if proj is not None:
        wo, bo = proj
        args += [wo, bo.reshape(1, -1)]
        in_specs.append(pl.BlockSpec(wo.shape, lambda i: (0, 0)))
        in_specs.append(pl.BlockSpec((1, dout), lambda i: (0, 0)))
    out_shape = [jax.ShapeDtypeStruct((n, dout), jnp.float32)]
    out_specs = [pl.BlockSpec((br, dout), lambda i: (i, 0))]
    if e_new:
        out_shape.append(jax.ShapeDtypeStruct((n, DM), jnp.float32))
        out_specs.append(pl.BlockSpec((br, DM), lambda i: (i, 0)))
    res = pl.pallas_call(
        body,
        grid=(pl.cdiv(n, br),),
        in_specs=in_specs,
        out_specs=out_specs,
        out_shape=out_shape,
    )(*args)
    return res if e_new else res[0]


def _sc_gather2(ta, tb, ia3, ib3):
    cpw = ia3.shape[1]
    nd = 3
    nbody, ntail = divmod(cpw, 2 * nd)
    e_pad = NW * cpw * 128
    mesh = plsc.VectorSubcoreMesh(core_axis_name="c", subcore_axis_name="s",
                                  num_cores=NC, num_subcores=NS)

    @functools.partial(
        pl.kernel,
        out_type=(jax.ShapeDtypeStruct((e_pad, DM), jnp.float32),
                  jax.ShapeDtypeStruct((e_pad, DM), jnp.float32)),
        mesh=mesh,
        scratch_types=[pltpu.VMEM((2, cpw, 128), jnp.int32),
                       pltpu.VMEM((2 * nd, 128, DM), jnp.float32)]
                      + [pltpu.SemaphoreType.DMA] * (4 * nd),
    )
    def k(ta_h, tb_h, ia_h, ib_h, oa_h, ob_h, idxv, rows, *sems):
        gsems, osems = sems[:2 * nd], sems[2 * nd:]
        wid = lax.axis_index("s") * NC + lax.axis_index("c")
        pltpu.sync_copy(ia_h.at[wid], idxv.at[0])
        pltpu.sync_copy(ib_h.at[wid], idxv.at[1])
        for t in range(2):
            tbl = ta_h if t == 0 else tb_h
            out = oa_h if t == 0 else ob_h

            def fire(j0, par, nwin, tbl=tbl, t=t):
                return [pltpu.async_copy(tbl.at[idxv.at[t, j0 + s]],
                                         rows.at[par * nd + s],
                                         gsems[par * nd + s])
                        for s in range(nwin)]

            def drain(j0, par, nwin, gds, out=out):
                ods = []
                for s in range(nwin):
                    gds[s].wait()
                    ods.append(pltpu.async_copy(
                        rows.at[par * nd + s],
                        out.at[pl.ds((wid * cpw + j0 + s) * 128, 128)],
                        osems[par * nd + s]))
                return ods

            def body(i, carry):
                j0 = i * 2 * nd
                ga = fire(j0, 0, nd)
                gb = fire(j0 + nd, 1, nd)
                oa = drain(j0, 0, nd, ga)
                ob = drain(j0 + nd, 1, nd, gb)
                for d in oa + ob:
                    d.wait()
                return carry

            lax.fori_loop(0, nbody, body, 0)
            for j in range(ntail):
                jj = nbody * 2 * nd + j
                g = fire(jj, 0, 1)
                for d in drain(jj, 0, 1, g):
                    d.wait()

    return k(ta, tb, ia3, ib3)


def _sc_scatter(m2, idx2, n_nodes, n_passes):
    cpt = idx2.shape[1]
    r_al = _rup(-(-n_nodes // (NC * n_passes)), 128)
    r_pad = _rup(r_al + 1, NS * 8)
    zr = r_pad // NS
    zc, zrem = divmod(zr, 128)
    wb = r_al // NS
    nd = 4
    nbody, ntail = divmod(cpt, nd)
    mesh = plsc.VectorSubcoreMesh(core_axis_name="c", subcore_axis_name="s",
                                  num_cores=NC, num_subcores=NS)

    @functools.partial(
        pl.kernel,
        out_type=jax.ShapeDtypeStruct((n_nodes, DM), jnp.float32),
        mesh=mesh,
        scratch_types=[pltpu.VMEM((cpt, 128), jnp.int32),
                       pltpu.VMEM((cpt, 128), jnp.int32),
                       pltpu.VMEM((nd, 128, DM), jnp.float32),
                       pltpu.VMEM_SHARED((r_pad, DM), jnp.float32)]
                      + [pltpu.SemaphoreType.DMA] * (2 * nd + 1),
    )
    def k(m_h, i_h, out_h, idxb, lidxb, rows, shared, *sems):
        lsems, ssems, zsem = sems[:nd], sems[nd:2 * nd], sems[2 * nd]
        cid = lax.axis_index("c")
        sid = lax.axis_index("s")
        pltpu.sync_copy(i_h.at[sid], idxb)
        zb = sid * zr
        for pss in range(n_passes):
            rs = jnp.minimum((cid * n_passes + pss) * r_al, n_nodes - r_al)

            def zrow(rr, carry):
                for j in range(8):
                    rows[0, rr, pl.ds(j * 16, 16)] = jnp.zeros((16,), jnp.float32)
                return carry
            lax.fori_loop(0, 128, zrow, 0)
            zds = [pltpu.async_copy(rows.at[0],
                                    shared.at[pl.ds(zb + z * 128, 128)], zsem)
                   for z in range(zc)]
            if zrem:
                zds.append(pltpu.async_copy(
                    rows.at[0, pl.ds(0, zrem)],
                    shared.at[pl.ds(zb + zc * 128, zrem)], zsem))

            def lix(c, carry):
                for j in range(8):
                    v = idxb[c, pl.ds(j * 16, 16)]
                    li = v - rs
                    okm = (li >= 0) & (li < r_al)
                    lidxb[c, pl.ds(j * 16, 16)] = jnp.where(okm, li, r_al)
                return carry
            lax.fori_loop(0, cpt, lix, 0)
            for d in zds:
                d.wait()
            plsc.subcore_barrier()

            def win(j0, nwin):
                lds = [pltpu.async_copy(
                           m_h.at[pl.ds((sid * cpt + j0 + s) * 128, 128)],
                           rows.at[s], lsems[s])
                       for s in range(nwin)]
                sds = []
                for s in range(nwin):
                    lds[s].wait()
                    sds.append(pltpu.async_copy(
                        rows.at[s], shared.at[lidxb.at[j0 + s]],
                        ssems[s], add=True))
                for s in range(nwin):
                    sds[s].wait()

            def body(i, carry):
                win(i * nd, nd)
                return carry

            lax.fori_loop(0, nbody, body, 0)
            if ntail:
                win(nbody * nd, ntail)
            plsc.subcore_barrier()
            pltpu.sync_copy(shared.at[pl.ds(sid * wb, wb)],
                            out_h.at[pl.ds(rs + sid * wb, wb)])
            plsc.subcore_barrier()

    return k(m2, idx2)


def _pad_rows(a, n_pad):
    e = a.shape[0]
    if e == n_pad:
        return a
    return jnp.concatenate(
        [a, jnp.zeros((n_pad - e,) + a.shape[1:], a.dtype)], axis=0)


def _pad_idx(idx, n_pad, fill):
    e = idx.shape[0]
    if e != n_pad:
        idx = jnp.concatenate(
            [idx, jnp.full((n_pad - e,), fill, jnp.int32)], axis=0)
    return idx


def _pad_idx_g(idx, n_pad):
    return _pad_idx(idx, n_pad, 0).reshape(NW, -1, 128)


def _pad_idx_s(idx, n_pad):
    return _pad_idx(idx, n_pad, 1 << 30).reshape(NS, -1, 128)


def _halves(n_e):
    e1 = min(_rup(n_e - n_e // 2, NW * 128), n_e)
    return [(0, e1, _rup(e1, NW * 128)), (e1, n_e - e1, _rup(n_e - e1, NW * 128))]


def _edge_idx(edge_index, halves):
    out = []
    for lo, cnt, pad in halves:
        s = lax.dynamic_slice_in_dim(edge_index[0], lo, cnt)
        d = lax.dynamic_slice_in_dim(edge_index[1], lo, cnt)
        out.append((_pad_idx_g(s, pad), _pad_idx_g(d, pad), _pad_idx_s(d, pad)))
    return out


def _mp_half(msg_p, ta, tb, e_half, idx3, n_dst, n_passes, e_new=False):
    gi_s, gi_d, si_d = idx3
    gs, gd = _sc_gather2(ta, tb, gi_s, gi_d)
    if e_new:
        m, e_out = _mlp(msg_p, [gs, gd, e_half], e_new=True)
    else:
        m = _mlp(msg_p, [gs, gd, e_half])
        e_out = None
    agg = _sc_scatter(m, si_d, n_dst, n_passes)
    return agg, e_out


def kernel(x, mgroupdef, e2h_edge_index, h2h_edge_index, h2e_edge_index,
           e2h_edge_attr, h2h_edge_attr, h2e_edge_attr,
           era_latlons, h_latlons, params):
    p = params
    bs = x.shape[0]
    n_era = x.shape[2]
    n_h = h_latlons.shape[0]
    e2h_h = _halves(e2h_edge_index.shape[1])
    h2h_h = _halves(h2h_edge_index.shape[1])
    h2e_h = _halves(h2e_edge_index.shape[1])

    def enc_edges(attr, tr, pe, halves):
        es = []
        for lo, cnt, pad in halves:
            a = _pad_rows(lax.dynamic_slice_in_dim(attr, lo, cnt), pad)
            t = _pad_rows(lax.dynamic_slice_in_dim(tr, lo, cnt), pad)
            es.append(_mlp(pe, [a, t]))
        return es

    x_flat = jnp.transpose(x, (0, 2, 1, 3)).reshape(bs * n_era, -1)

    src = _mlp(p["fm_src"], [x_flat, era_latlons, p["era_trainable"]])
    dst = _mlp(p["fm_dst"], [h_latlons, p["h_trainable"]])
    e_fm = enc_edges(e2h_edge_attr, p["e2h_trainable"], p["fm_edge"], e2h_h)
    e_pr = enc_edges(h2h_edge_attr, p["h2h_trainable"], p["proc_edge"], h2h_h)
    e_bm = enc_edges(h2e_edge_attr, p["h2e_trainable"], p["bm_edge"], h2e_h)

    nodeoffs = [0, DM, DM]

    e2h_i = _edge_idx(e2h_edge_index, e2h_h)
    agg0, _ = _mp_half(p["fm_msg"], src, dst, e_fm[0], e2h_i[0], n_h, 1)
    agg1, _ = _mp_half(p["fm_msg"], src, dst, e_fm[1], e2h_i[1], n_h, 1)
    x_latent = _mlp(p["fm_node"], [dst, agg0, agg1], residual=True,
                    offs=nodeoffs)

    h2h_i = _edge_idx(h2h_edge_index, h2h_h)
    agg0, e0 = _mp_half(p["proc_msg_0"], x_latent, x_latent, e_pr[0],
                        h2h_i[0], n_h, 1, e_new=True)
    agg1, e1 = _mp_half(p["proc_msg_0"], x_latent, x_latent, e_pr[1],
                        h2h_i[1], n_h, 1, e_new=True)
    x_latent = _mlp(p["proc_node_0"], [x_latent, agg0, agg1], residual=True,
                    offs=nodeoffs)
    agg0, _ = _mp_half(p["proc_msg_1"], x_latent, x_latent, e0, h2h_i[0], n_h, 1)
    agg1, _ = _mp_half(p["proc_msg_1"], x_latent, x_latent, e1, h2h_i[1], n_h, 1)
    x_latent = _mlp(p["proc_node_1"], [x_latent, agg0, agg1], residual=True,
                    offs=nodeoffs)

    h2e_i = _edge_idx(h2e_edge_index, h2e_h)
    agg0, _ = _mp_half(p["bm_msg"], x_latent, src, e_bm[0], h2e_i[0], n_era, 5)
    agg1, _ = _mp_half(p["bm_msg"], x_latent, src, e_bm[1], h2e_i[1], n_era, 5)
    out = _mlp(p["bm_node"], [src, agg0, agg1], residual=True,
               proj=(p["bm_out_W"], p["bm_out_b"]), offs=nodeoffs)
    return out.reshape(bs, n_era, -1)


# speedup vs baseline: 1.1536x; 1.0111x over previous
import functools

import jax
import jax.numpy as jnp
from jax import lax
from jax.experimental import pallas as pl
from jax.experimental.pallas import tpu as pltpu
from jax.experimental.pallas import tpu_sc as plsc

NC, NS = 2, 16
NW = NC * NS
DM = 128


def _rup(n, m):
    return (n + m - 1) // m * m


def _mlp(p, xs, *, residual=False, e_new=False, proj=None, br=1024, offs=None):
    n = xs[0].shape[0]
    dins = [x.shape[1] for x in xs]
    k = len(xs)
    dout = proj[0].shape[1] if proj is not None else DM

    def body(*refs):
        xrefs = refs[:k]
        w1, b1, w2, b2, g, bn = refs[k:k + 6]
        pos = k + 6
        if proj is not None:
            wo, bo = refs[pos:pos + 2]
            pos += 2
        outs = refs[pos:]
        bf = jnp.bfloat16
        acc = None
        off = 0
        for i in range(k):
            o = offs[i] if offs is not None else off
            part = jnp.dot(xrefs[i][...].astype(bf),
                           w1[o:o + dins[i], :].astype(bf),
                           preferred_element_type=jnp.float32)
            acc = part if acc is None else acc + part
            off += dins[i]
        h = acc + b1[...]
        h = h * jax.nn.sigmoid(h)
        y = jnp.dot(h.astype(bf), w2[...].astype(bf),
                    preferred_element_type=jnp.float32) + b2[...]
        mu = jnp.mean(y, -1, keepdims=True)
        yc = y - mu
        var = jnp.mean(yc * yc, -1, keepdims=True)
        m = yc * lax.rsqrt(var + 1e-5) * g[...] + bn[...]
        r = xrefs[0][...] + m if residual else m
        if proj is not None:
            r = jnp.dot(r, wo[...], preferred_element_type=jnp.float32) + bo[...]
        outs[0][...] = r
        if e_new:
            outs[1][...] = xrefs[-1][...] + m

    in_specs = [pl.BlockSpec((br, d), lambda i: (i, 0)) for d in dins]
    w_args = [p["W1"], p["b1"].reshape(1, DM), p["W2"], p["b2"].reshape(1, DM),
              p["g"].reshape(1, DM), p["bn"].reshape(1, DM)]
    for w in w_args:
        in_specs.append(pl.BlockSpec(w.shape, lambda i: (0, 0)))
    args = list(xs) + w_args
    if proj is not None:
        wo, bo = proj
        args += [wo, bo.reshape(1, -1)]
        in_specs.append(pl.BlockSpec(wo.shape, lambda i: (0, 0)))
        in_specs.append(pl.BlockSpec((1, dout), lambda i: (0, 0)))
    out_shape = [jax.ShapeDtypeStruct((n, dout), jnp.float32)]
    out_specs = [pl.BlockSpec((br, dout), lambda i: (i, 0))]
    if e_new:
        out_shape.append(jax.ShapeDtypeStruct((n, DM), jnp.float32))
        out_specs.append(pl.BlockSpec((br, DM), lambda i: (i, 0)))
    res = pl.pallas_call(
        body,
        grid=(pl.cdiv(n, br),),
        in_specs=in_specs,
        out_specs=out_specs,
        out_shape=out_shape,
    )(*args)
    return res if e_new else res[0]


def _sc_gather2(ta, tb, ia3, ib3):
    cpw = ia3.shape[1]
    nd = 3
    nbody, ntail = divmod(cpw, 2 * nd)
    e_pad = NW * cpw * 128
    mesh = plsc.VectorSubcoreMesh(core_axis_name="c", subcore_axis_name="s",
                                  num_cores=NC, num_subcores=NS)

    @functools.partial(
        pl.kernel,
        out_type=(jax.ShapeDtypeStruct((e_pad, DM), jnp.float32),
                  jax.ShapeDtypeStruct((e_pad, DM), jnp.float32)),
        mesh=mesh,
        scratch_types=[pltpu.VMEM((2, cpw, 128), jnp.int32),
                       pltpu.VMEM((2 * nd, 128, DM), jnp.float32)]
                      + [pltpu.SemaphoreType.DMA] * (4 * nd),
    )
    def k(ta_h, tb_h, ia_h, ib_h, oa_h, ob_h, idxv, rows, *sems):
        gsems, osems = sems[:2 * nd], sems[2 * nd:]
        wid = lax.axis_index("s") * NC + lax.axis_index("c")
        pltpu.sync_copy(ia_h.at[wid], idxv.at[0])
        pltpu.sync_copy(ib_h.at[wid], idxv.at[1])
        for t in range(2):
            tbl = ta_h if t == 0 else tb_h
            out = oa_h if t == 0 else ob_h

            def fire(j0, par, nwin, tbl=tbl, t=t):
                return [pltpu.async_copy(tbl.at[idxv.at[t, j0 + s]],
                                         rows.at[par * nd + s],
                                         gsems[par * nd + s])
                        for s in range(nwin)]

            def drain(j0, par, nwin, gds, out=out):
                ods = []
                for s in range(nwin):
                    gds[s].wait()
                    ods.append(pltpu.async_copy(
                        rows.at[par * nd + s],
                        out.at[pl.ds((wid * cpw + j0 + s) * 128, 128)],
                        osems[par * nd + s]))
                return ods

            def body(i, carry):
                j0 = i * 2 * nd
                ga = fire(j0, 0, nd)
                gb = fire(j0 + nd, 1, nd)
                oa = drain(j0, 0, nd, ga)
                ob = drain(j0 + nd, 1, nd, gb)
                for d in oa + ob:
                    d.wait()
                return carry

            lax.fori_loop(0, nbody, body, 0)
            for j in range(ntail):
                jj = nbody * 2 * nd + j
                g = fire(jj, 0, 1)
                for d in drain(jj, 0, 1, g):
                    d.wait()

    return k(ta, tb, ia3, ib3)


def _sc_scatter(m2, idx2, n_nodes, n_passes):
    cpt = idx2.shape[1]
    r_al = _rup(-(-n_nodes // (NC * n_passes)), 128)
    r_pad = _rup(r_al + 1, NS * 8)
    zr = r_pad // NS
    zc, zrem = divmod(zr, 128)
    wb = r_al // NS
    nd = 4
    nbody, ntail = divmod(cpt, nd)
    mesh = plsc.VectorSubcoreMesh(core_axis_name="c", subcore_axis_name="s",
                                  num_cores=NC, num_subcores=NS)

    @functools.partial(
        pl.kernel,
        out_type=jax.ShapeDtypeStruct((n_nodes, DM), jnp.float32),
        mesh=mesh,
        scratch_types=[pltpu.VMEM((cpt, 128), jnp.int32),
                       pltpu.VMEM((cpt, 128), jnp.int32),
                       pltpu.VMEM((nd, 128, DM), jnp.float32),
                       pltpu.VMEM_SHARED((r_pad, DM), jnp.float32)]
                      + [pltpu.SemaphoreType.DMA] * (2 * nd + 1),
    )
    def k(m_h, i_h, out_h, idxb, lidxb, rows, shared, *sems):
        lsems, ssems, zsem = sems[:nd], sems[nd:2 * nd], sems[2 * nd]
        cid = lax.axis_index("c")
        sid = lax.axis_index("s")
        pltpu.sync_copy(i_h.at[sid], idxb)
        zb = sid * zr
        for pss in range(n_passes):
            rs = jnp.minimum((cid * n_passes + pss) * r_al, n_nodes - r_al)

            def zrow(rr, carry):
                for j in range(8):
                    rows[0, rr, pl.ds(j * 16, 16)] = jnp.zeros((16,), jnp.float32)
                return carry
            lax.fori_loop(0, 128, zrow, 0)
            zds = [pltpu.async_copy(rows.at[0],
                                    shared.at[pl.ds(zb + z * 128, 128)], zsem)
                   for z in range(zc)]
            if zrem:
                zds.append(pltpu.async_copy(
                    rows.at[0, pl.ds(0, zrem)],
                    shared.at[pl.ds(zb + zc * 128, zrem)], zsem))

            dummy = r_al + sid * 4
            def lix(c, carry):
                for j in range(8):
                    v = idxb[c, pl.ds(j * 16, 16)]
                    li = v - rs
                    okm = (li >= 0) & (li < r_al)
                    lidxb[c, pl.ds(j * 16, 16)] = jnp.where(okm, li, dummy)
                return carry
            lax.fori_loop(0, cpt, lix, 0)
            for d in zds:
                d.wait()
            plsc.subcore_barrier()

            def win(j0, nwin):
                lds = [pltpu.async_copy(
                           m_h.at[pl.ds((sid * cpt + j0 + s) * 128, 128)],
                           rows.at[s], lsems[s])
                       for s in range(nwin)]
                sds = []
                for s in range(nwin):
                    lds[s].wait()
                    sds.append(pltpu.async_copy(
                        rows.at[s], shared.at[lidxb.at[j0 + s]],
                        ssems[s], add=True))
                for s in range(nwin):
                    sds[s].wait()

            def body(i, carry):
                win(i * nd, nd)
                return carry

            lax.fori_loop(0, nbody, body, 0)
            if ntail:
                win(nbody * nd, ntail)
            plsc.subcore_barrier()
            pltpu.sync_copy(shared.at[pl.ds(sid * wb, wb)],
                            out_h.at[pl.ds(rs + sid * wb, wb)])
            plsc.subcore_barrier()

    return k(m2, idx2)


def _pad_rows(a, n_pad):
    e = a.shape[0]
    if e == n_pad:
        return a
    return jnp.concatenate(
        [a, jnp.zeros((n_pad - e,) + a.shape[1:], a.dtype)], axis=0)


def _pad_idx(idx, n_pad, fill):
    e = idx.shape[0]
    if e != n_pad:
        idx = jnp.concatenate(
            [idx, jnp.full((n_pad - e,), fill, jnp.int32)], axis=0)
    return idx


def _pad_idx_g(idx, n_pad):
    return _pad_idx(idx, n_pad, 0).reshape(NW, -1, 128)


def _pad_idx_s(idx, n_pad):
    return _pad_idx(idx, n_pad, 1 << 30).reshape(NS, -1, 128)


def _halves(n_e):
    e1 = min(_rup(n_e - n_e // 2, NW * 128), n_e)
    return [(0, e1, _rup(e1, NW * 128)), (e1, n_e - e1, _rup(n_e - e1, NW * 128))]


def _edge_idx(edge_index, halves):
    out = []
    for lo, cnt, pad in halves:
        s = lax.dynamic_slice_in_dim(edge_index[0], lo, cnt)
        d = lax.dynamic_slice_in_dim(edge_index[1], lo, cnt)
        out.append((_pad_idx_g(s, pad), _pad_idx_g(d, pad), _pad_idx_s(d, pad)))
    return out


def _mp_half(msg_p, ta, tb, e_half, idx3, n_dst, n_passes, e_new=False):
    gi_s, gi_d, si_d = idx3
    gs, gd = _sc_gather2(ta, tb, gi_s, gi_d)
    if e_new:
        m, e_out = _mlp(msg_p, [gs, gd, e_half], e_new=True)
    else:
        m = _mlp(msg_p, [gs, gd, e_half])
        e_out = None
    agg = _sc_scatter(m, si_d, n_dst, n_passes)
    return agg, e_out


def kernel(x, mgroupdef, e2h_edge_index, h2h_edge_index, h2e_edge_index,
           e2h_edge_attr, h2h_edge_attr, h2e_edge_attr,
           era_latlons, h_latlons, params):
    p = params
    bs = x.shape[0]
    n_era = x.shape[2]
    n_h = h_latlons.shape[0]
    e2h_h = _halves(e2h_edge_index.shape[1])
    h2h_h = _halves(h2h_edge_index.shape[1])
    h2e_h = _halves(h2e_edge_index.shape[1])

    def enc_edges(attr, tr, pe, halves):
        es = []
        for lo, cnt, pad in halves:
            a = _pad_rows(lax.dynamic_slice_in_dim(attr, lo, cnt), pad)
            t = _pad_rows(lax.dynamic_slice_in_dim(tr, lo, cnt), pad)
            es.append(_mlp(pe, [a, t]))
        return es

    x_flat = jnp.transpose(x, (0, 2, 1, 3)).reshape(bs * n_era, -1)

    src = _mlp(p["fm_src"], [x_flat, era_latlons, p["era_trainable"]])
    dst = _mlp(p["fm_dst"], [h_latlons, p["h_trainable"]])
    e_fm = enc_edges(e2h_edge_attr, p["e2h_trainable"], p["fm_edge"], e2h_h)
    e_pr = enc_edges(h2h_edge_attr, p["h2h_trainable"], p["proc_edge"], h2h_h)
    e_bm = enc_edges(h2e_edge_attr, p["h2e_trainable"], p["bm_edge"], h2e_h)

    nodeoffs = [0, DM, DM]

    e2h_i = _edge_idx(e2h_edge_index, e2h_h)
    agg0, _ = _mp_half(p["fm_msg"], src, dst, e_fm[0], e2h_i[0], n_h, 1)
    agg1, _ = _mp_half(p["fm_msg"], src, dst, e_fm[1], e2h_i[1], n_h, 1)
    x_latent = _mlp(p["fm_node"], [dst, agg0, agg1], residual=True,
                    offs=nodeoffs)

    h2h_i = _edge_idx(h2h_edge_index, h2h_h)
    agg0, e0 = _mp_half(p["proc_msg_0"], x_latent, x_latent, e_pr[0],
                        h2h_i[0], n_h, 1, e_new=True)
    agg1, e1 = _mp_half(p["proc_msg_0"], x_latent, x_latent, e_pr[1],
                        h2h_i[1], n_h, 1, e_new=True)
    x_latent = _mlp(p["proc_node_0"], [x_latent, agg0, agg1], residual=True,
                    offs=nodeoffs)
    agg0, _ = _mp_half(p["proc_msg_1"], x_latent, x_latent, e0, h2h_i[0], n_h, 1)
    agg1, _ = _mp_half(p["proc_msg_1"], x_latent, x_latent, e1, h2h_i[1], n_h, 1)
    x_latent = _mlp(p["proc_node_1"], [x_latent, agg0, agg1], residual=True,
                    offs=nodeoffs)

    h2e_i = _edge_idx(h2e_edge_index, h2e_h)
    agg0, _ = _mp_half(p["bm_msg"], x_latent, src, e_bm[0], h2e_i[0], n_era, 5)
    agg1, _ = _mp_half(p["bm_msg"], x_latent, src, e_bm[1], h2e_i[1], n_era, 5)
    out = _mlp(p["bm_node"], [src, agg0, agg1], residual=True,
               proj=(p["bm_out_W"], p["bm_out_b"]), offs=nodeoffs)
    return out.reshape(bs, n_era, -1)


# speedup vs baseline: 1.2087x; 1.0477x over previous
import functools

import jax
import jax.numpy as jnp
from jax import lax
from jax.experimental import pallas as pl
from jax.experimental.pallas import tpu as pltpu
from jax.experimental.pallas import tpu_sc as plsc

NC, NS = 2, 16
NW = NC * NS
DM = 128


def _rup(n, m):
    return (n + m - 1) // m * m


def _mlp(p, xs, *, residual=False, e_new=False, proj=None, br=1024, offs=None):
    n = xs[0].shape[0]
    dins = [x.shape[1] for x in xs]
    k = len(xs)
    dout = proj[0].shape[1] if proj is not None else DM

    def body(*refs):
        xrefs = refs[:k]
        w1, b1, w2, b2, g, bn = refs[k:k + 6]
        pos = k + 6
        if proj is not None:
            wo, bo = refs[pos:pos + 2]
            pos += 2
        outs = refs[pos:]
        bf = jnp.bfloat16
        acc = None
        off = 0
        for i in range(k):
            o = offs[i] if offs is not None else off
            part = jnp.dot(xrefs[i][...].astype(bf),
                           w1[o:o + dins[i], :].astype(bf),
                           preferred_element_type=jnp.float32)
            acc = part if acc is None else acc + part
            off += dins[i]
        h = acc + b1[...]
        h = h * jax.nn.sigmoid(h)
        y = jnp.dot(h.astype(bf), w2[...].astype(bf),
                    preferred_element_type=jnp.float32) + b2[...]
        mu = jnp.mean(y, -1, keepdims=True)
        yc = y - mu
        var = jnp.mean(yc * yc, -1, keepdims=True)
        m = yc * lax.rsqrt(var + 1e-5) * g[...] + bn[...]
        r = xrefs[0][...] + m if residual else m
        if proj is not None:
            r = jnp.dot(r, wo[...], preferred_element_type=jnp.float32) + bo[...]
        outs[0][...] = r
        if e_new:
            outs[1][...] = xrefs[-1][...] + m

    in_specs = [pl.BlockSpec((br, d), lambda i: (i, 0)) for d in dins]
    w_args = [p["W1"], p["b1"].reshape(1, DM), p["W2"], p["b2"].reshape(1, DM),
              p["g"].reshape(1, DM), p["bn"].reshape(1, DM)]
    for w in w_args:
        in_specs.append(pl.BlockSpec(w.shape, lambda i: (0, 0)))
    args = list(xs) + w_args
    if proj is not None:
        wo, bo = proj
        args += [wo, bo.reshape(1, -1)]
        in_specs.append(pl.BlockSpec(wo.shape, lambda i: (0, 0)))
        in_specs.append(pl.BlockSpec((1, dout), lambda i: (0, 0)))
    out_shape = [jax.ShapeDtypeStruct((n, dout), jnp.float32)]
    out_specs = [pl.BlockSpec((br, dout), lambda i: (i, 0))]
    if e_new:
        out_shape.append(jax.ShapeDtypeStruct((n, DM), jnp.float32))
        out_specs.append(pl.BlockSpec((br, DM), lambda i: (i, 0)))
    res = pl.pallas_call(
        body,
        grid=(pl.cdiv(n, br),),
        in_specs=in_specs,
        out_specs=out_specs,
        out_shape=out_shape,
    )(*args)
    return res if e_new else res[0]


def _sc_gather2(ta, tb, ia3, ib3):
    cpw = ia3.shape[1]
    nd = 3
    nbody, ntail = divmod(cpw, 2 * nd)
    e_pad = NW * cpw * 128
    mesh = plsc.VectorSubcoreMesh(core_axis_name="c", subcore_axis_name="s",
                                  num_cores=NC, num_subcores=NS)

    @functools.partial(
        pl.kernel,
        out_type=(jax.ShapeDtypeStruct((e_pad, DM), jnp.float32),
                  jax.ShapeDtypeStruct((e_pad, DM), jnp.float32)),
        mesh=mesh,
        scratch_types=[pltpu.VMEM((2, cpw, 128), jnp.int32),
                       pltpu.VMEM((2 * nd, 128, DM), jnp.float32)]
                      + [pltpu.SemaphoreType.DMA] * (4 * nd),
    )
    def k(ta_h, tb_h, ia_h, ib_h, oa_h, ob_h, idxv, rows, *sems):
        gsems, osems = sems[:2 * nd], sems[2 * nd:]
        wid = lax.axis_index("s") * NC + lax.axis_index("c")
        pltpu.sync_copy(ia_h.at[wid], idxv.at[0])
        pltpu.sync_copy(ib_h.at[wid], idxv.at[1])
        for t in range(2):
            tbl = ta_h if t == 0 else tb_h
            out = oa_h if t == 0 else ob_h

            def fire(j0, par, nwin, tbl=tbl, t=t):
                return [pltpu.async_copy(tbl.at[idxv.at[t, j0 + s]],
                                         rows.at[par * nd + s],
                                         gsems[par * nd + s])
                        for s in range(nwin)]

            def drain(j0, par, nwin, gds, out=out):
                ods = []
                for s in range(nwin):
                    gds[s].wait()
                    ods.append(pltpu.async_copy(
                        rows.at[par * nd + s],
                        out.at[pl.ds((wid * cpw + j0 + s) * 128, 128)],
                        osems[par * nd + s]))
                return ods

            def body(i, carry):
                j0 = i * 2 * nd
                ga = fire(j0, 0, nd)
                gb = fire(j0 + nd, 1, nd)
                oa = drain(j0, 0, nd, ga)
                ob = drain(j0 + nd, 1, nd, gb)
                for d in oa + ob:
                    d.wait()
                return carry

            lax.fori_loop(0, nbody, body, 0)
            for j in range(ntail):
                jj = nbody * 2 * nd + j
                g = fire(jj, 0, 1)
                for d in drain(jj, 0, 1, g):
                    d.wait()

    return k(ta, tb, ia3, ib3)


def _sc_scatter(m2, idx2, n_nodes, n_passes):
    cpt = idx2.shape[1]
    r_al = _rup(-(-n_nodes // (NC * n_passes)), 128)
    r_pad = _rup(r_al + 1, NS * 8)
    zr = r_pad // NS
    zc, zrem = divmod(zr, 128)
    wb = r_al // NS
    nd = 4
    nbody, ntail = divmod(cpt, nd)
    mesh = plsc.VectorSubcoreMesh(core_axis_name="c", subcore_axis_name="s",
                                  num_cores=NC, num_subcores=NS)

    @functools.partial(
        pl.kernel,
        out_type=jax.ShapeDtypeStruct((n_nodes, DM), jnp.float32),
        mesh=mesh,
        scratch_types=[pltpu.VMEM((cpt, 128), jnp.int32),
                       pltpu.VMEM((cpt, 128), jnp.int32),
                       pltpu.VMEM((nd, 128, DM), jnp.float32),
                       pltpu.VMEM_SHARED((r_pad, DM), jnp.float32)]
                      + [pltpu.SemaphoreType.DMA] * (2 * nd + 1),
    )
    def k(m_h, i_h, out_h, idxb, lidxb, rows, shared, *sems):
        lsems, ssems, zsem = sems[:nd], sems[nd:2 * nd], sems[2 * nd]
        cid = lax.axis_index("c")
        sid = lax.axis_index("s")
        pltpu.sync_copy(i_h.at[sid], idxb)
        zb = sid * zr
        for pss in range(n_passes):
            rs = jnp.minimum((cid * n_passes + pss) * r_al, n_nodes - r_al)

            def zrow(rr, carry):
                for j in range(8):
                    rows[0, rr, pl.ds(j * 16, 16)] = jnp.zeros((16,), jnp.float32)
                return carry
            lax.fori_loop(0, 128, zrow, 0)
            zds = [pltpu.async_copy(rows.at[0],
                                    shared.at[pl.ds(zb + z * 128, 128)], zsem)
                   for z in range(zc)]
            if zrem:
                zds.append(pltpu.async_copy(
                    rows.at[0, pl.ds(0, zrem)],
                    shared.at[pl.ds(zb + zc * 128, zrem)], zsem))

            dummy = r_al + sid * 4
            def lix(c, carry):
                for j in range(8):
                    v = idxb[c, pl.ds(j * 16, 16)]
                    li = v - rs
                    okm = (li >= 0) & (li < r_al)
                    lidxb[c, pl.ds(j * 16, 16)] = jnp.where(okm, li, dummy)
                return carry
            lax.fori_loop(0, cpt, lix, 0)
            for d in zds:
                d.wait()
            plsc.subcore_barrier()

            def win(j0, nwin):
                lds = [pltpu.async_copy(
                           m_h.at[pl.ds((sid * cpt + j0 + s) * 128, 128)],
                           rows.at[s], lsems[s])
                       for s in range(nwin)]
                sds = []
                for s in range(nwin):
                    lds[s].wait()
                    sds.append(pltpu.async_copy(
                        rows.at[s], shared.at[lidxb.at[j0 + s]],
                        ssems[s], add=True))
                for s in range(nwin):
                    sds[s].wait()

            def body(i, carry):
                win(i * nd, nd)
                return carry

            lax.fori_loop(0, nbody, body, 0)
            if ntail:
                win(nbody * nd, ntail)
            plsc.subcore_barrier()
            pltpu.sync_copy(shared.at[pl.ds(sid * wb, wb)],
                            out_h.at[pl.ds(rs + sid * wb, wb)])
            plsc.subcore_barrier()

    return k(m2, idx2)


def _pad_rows(a, n_pad):
    e = a.shape[0]
    if e == n_pad:
        return a
    return jnp.concatenate(
        [a, jnp.zeros((n_pad - e,) + a.shape[1:], a.dtype)], axis=0)


def _pad_idx(idx, n_pad, fill):
    e = idx.shape[0]
    if e != n_pad:
        idx = jnp.concatenate(
            [idx, jnp.full((n_pad - e,), fill, jnp.int32)], axis=0)
    return idx


def _pad_idx_g(idx, n_pad):
    return _pad_idx(idx, n_pad, 0).reshape(NW, -1, 128)


def _pad_idx_s(idx, n_pad):
    return _pad_idx(idx, n_pad, 1 << 30).reshape(NS, -1, 128)


def _halves(n_e):
    e1 = min(_rup(n_e - n_e // 2, NW * 128), n_e)
    return [(0, e1, _rup(e1, NW * 128)), (e1, n_e - e1, _rup(n_e - e1, NW * 128))]


def _edge_idx(edge_index, halves):
    out = []
    for lo, cnt, pad in halves:
        s = lax.dynamic_slice_in_dim(edge_index[0], lo, cnt)
        d = lax.dynamic_slice_in_dim(edge_index[1], lo, cnt)
        out.append((_pad_idx_g(s, pad), _pad_idx_g(d, pad), _pad_idx_s(d, pad)))
    return out


def _mp_half(msg_p, ta, tb, e_half, idx3, n_dst, n_passes, e_new=False):
    gi_s, gi_d, si_d = idx3
    gs, gd = _sc_gather2(ta, tb, gi_s, gi_d)
    if e_new:
        m, e_out = _mlp(msg_p, [gs, gd, e_half], e_new=True)
    else:
        m = _mlp(msg_p, [gs, gd, e_half])
        e_out = None
    agg = _sc_scatter(m, si_d, n_dst, n_passes)
    return agg, e_out


def kernel(x, mgroupdef, e2h_edge_index, h2h_edge_index, h2e_edge_index,
           e2h_edge_attr, h2h_edge_attr, h2e_edge_attr,
           era_latlons, h_latlons, params):
    p = params
    bs = x.shape[0]
    n_era = x.shape[2]
    n_h = h_latlons.shape[0]
    e2h_h = _halves(e2h_edge_index.shape[1])
    h2h_h = _halves(h2h_edge_index.shape[1])
    h2e_h = _halves(h2e_edge_index.shape[1])

    def enc_edges(attr, tr, pe, halves):
        es = []
        for lo, cnt, pad in halves:
            a = _pad_rows(lax.dynamic_slice_in_dim(attr, lo, cnt), pad)
            t = _pad_rows(lax.dynamic_slice_in_dim(tr, lo, cnt), pad)
            es.append(_mlp(pe, [a, t]))
        return es

    x_flat = jnp.transpose(x, (0, 2, 1, 3)).reshape(bs * n_era, -1)

    src = _mlp(p["fm_src"], [x_flat, era_latlons, p["era_trainable"]])
    dst = _mlp(p["fm_dst"], [h_latlons, p["h_trainable"]])
    e_fm = enc_edges(e2h_edge_attr, p["e2h_trainable"], p["fm_edge"], e2h_h)
    e_pr = enc_edges(h2h_edge_attr, p["h2h_trainable"], p["proc_edge"], h2h_h)
    e_bm = enc_edges(h2e_edge_attr, p["h2e_trainable"], p["bm_edge"], h2e_h)

    nodeoffs = [0, DM, DM]

    e2h_i = _edge_idx(e2h_edge_index, e2h_h)
    agg0, _ = _mp_half(p["fm_msg"], src, dst, e_fm[0], e2h_i[0], n_h, 1)
    agg1, _ = _mp_half(p["fm_msg"], src, dst, e_fm[1], e2h_i[1], n_h, 1)
    x_latent = _mlp(p["fm_node"], [dst, agg0, agg1], residual=True,
                    offs=nodeoffs)

    h2h_i = _edge_idx(h2h_edge_index, h2h_h)
    agg0, e0 = _mp_half(p["proc_msg_0"], x_latent, x_latent, e_pr[0],
                        h2h_i[0], n_h, 1, e_new=True)
    agg1, e1 = _mp_half(p["proc_msg_0"], x_latent, x_latent, e_pr[1],
                        h2h_i[1], n_h, 1, e_new=True)
    x_latent = _mlp(p["proc_node_0"], [x_latent, agg0, agg1], residual=True,
                    offs=nodeoffs)
    agg0, _ = _mp_half(p["proc_msg_1"], x_latent, x_latent, e0, h2h_i[0], n_h, 1)
    agg1, _ = _mp_half(p["proc_msg_1"], x_latent, x_latent, e1, h2h_i[1], n_h, 1)
    x_latent = _mlp(p["proc_node_1"], [x_latent, agg0, agg1], residual=True,
                    offs=nodeoffs)

    h2e_i = _edge_idx(h2e_edge_index, h2e_h)
    agg0, _ = _mp_half(p["bm_msg"], x_latent, src, e_bm[0], h2e_i[0], n_era, 4)
    agg1, _ = _mp_half(p["bm_msg"], x_latent, src, e_bm[1], h2e_i[1], n_era, 4)
    out = _mlp(p["bm_node"], [src, agg0, agg1], residual=True,
               proj=(p["bm_out_W"], p["bm_out_b"]), offs=nodeoffs)
    return out.reshape(bs, n_era, -1)
